# baseline (speedup 1.0000x reference)
import functools

import jax
import jax.numpy as jnp
from jax import lax
from jax.experimental import pallas as pl
from jax.experimental.pallas import tpu as pltpu
from jax.experimental.pallas import tpu_sc as plsc

T = 2048
D = 2048
E = 8
TOPK = 2
I = 2048

S = T * TOPK
M = 1024
IB = 256
NT = S // M
NV = NT + E - 1
NJ = I // IB
R = 512

_MESH = plsc.VectorSubcoreMesh(core_axis_name="c", subcore_axis_name="s")
NC = 2
NS = 16
NW = NC * NS


def _router_body(logits_ref, pos1_ref, pos2_ref, counts_ref,
                 wb1_ref, wb2_ref):
    logits = logits_ref[...]
    cols = lax.broadcasted_iota(jnp.int32, (T, E), 1)
    m1 = jnp.max(logits, axis=1, keepdims=True)
    i1 = jnp.min(jnp.where(logits == m1, cols, E), axis=1, keepdims=True)
    masked = jnp.where(cols == i1, -jnp.inf, logits)
    m2 = jnp.max(masked, axis=1, keepdims=True)
    i2 = jnp.min(jnp.where(masked == m2, cols, E), axis=1, keepdims=True)
    w1 = 1.0 / (1.0 + jnp.exp(m2 - m1))
    ones16 = jnp.ones((T, 16), jnp.float32)
    wb1_ref[...] = w1 * ones16
    wb2_ref[...] = (1.0 - w1) * ones16

    mA = jnp.where(cols == i1, 1.0, 0.0)
    mB = jnp.where(cols == i2, 1.0, 0.0)
    counts = (jnp.sum(mA, axis=0, keepdims=True)
              + jnp.sum(mB, axis=0, keepdims=True))
    counts_ref[...] = counts.astype(jnp.int32)
    er = lax.broadcasted_iota(jnp.int32, (E, E), 0)
    ec = lax.broadcasted_iota(jnp.int32, (E, E), 1)
    lt_e = jnp.where(er < ec, 1.0, 0.0)
    offs = lax.dot_general(counts, lt_e, (((1,), (0,)), ((), ())),
                           precision=lax.Precision.HIGHEST,
                           preferred_element_type=jnp.float32)

    rr = lax.broadcasted_iota(jnp.int32, (R, R), 0)
    rc = lax.broadcasted_iota(jnp.int32, (R, R), 1)
    ltri = jnp.where(rr > rc, 1.0, 0.0)

    base = jnp.zeros((1, E), jnp.float32)
    for mat, pref in ((mA, pos1_ref), (mB, pos2_ref)):
        for ci in range(T // R):
            mc = lax.slice(mat, (ci * R, 0), (ci * R + R, E))
            prior = lax.dot_general(ltri, mc, (((1,), (0,)), ((), ())),
                                    precision=lax.Precision.HIGHEST,
                                    preferred_element_type=jnp.float32)
            posf = jnp.sum((offs + base + prior) * mc, axis=1, keepdims=True)
            pref[pl.ds(ci * R, R), :] = posf.astype(jnp.int32)
            base = base + jnp.sum(mc, axis=0, keepdims=True)


def _router(logits):
    return pl.pallas_call(
        _router_body,
        out_shape=[
            jax.ShapeDtypeStruct((T, 1), jnp.int32),
            jax.ShapeDtypeStruct((T, 1), jnp.int32),
            jax.ShapeDtypeStruct((1, E), jnp.int32),
            jax.ShapeDtypeStruct((T, 16), jnp.float32),
            jax.ShapeDtypeStruct((T, 16), jnp.float32),
        ],
    )(logits)


_DCH = 32
_DN = T // (NW * _DCH)


@functools.partial(
    pl.kernel,
    mesh=_MESH,
    out_type=jax.ShapeDtypeStruct((S, D), jnp.float32),
    scratch_types=[
        pltpu.VMEM((_DCH,), jnp.int32),
        pltpu.VMEM((_DCH,), jnp.int32),
        pltpu.VMEM((_DCH, D), jnp.float32),
        pltpu.SemaphoreType.DMA,
        pltpu.SemaphoreType.DMA,
    ],
)
def _dispatch_kernel(x_hbm, pos1_hbm, pos2_hbm, out_hbm,
                     idx1_v, idx2_v, rows_v, sem1, sem2):
    wid = lax.axis_index("c") * NS + lax.axis_index("s")
    for ch in range(_DN):
        tb = pl.multiple_of(wid * (_DCH * _DN) + ch * _DCH, _DCH)
        pltpu.sync_copy(pos1_hbm.at[pl.ds(tb, _DCH)], idx1_v)
        pltpu.sync_copy(pos2_hbm.at[pl.ds(tb, _DCH)], idx2_v)
        pltpu.sync_copy(x_hbm.at[pl.ds(tb, _DCH)], rows_v)
        cp1 = pltpu.async_copy(rows_v, out_hbm.at[idx1_v], sem1)
        cp2 = pltpu.async_copy(rows_v, out_hbm.at[idx2_v], sem2)
        cp1.wait()
        cp2.wait()


_CCH = 16
_CN = T // (NW * _CCH)


@functools.partial(
    pl.kernel,
    mesh=_MESH,
    out_type=jax.ShapeDtypeStruct((T, D), jnp.float32),
    scratch_types=[
        pltpu.VMEM((_CCH,), jnp.int32),
        pltpu.VMEM((_CCH,), jnp.int32),
        pltpu.VMEM((_CCH, D), jnp.float32),
        pltpu.VMEM((_CCH, D), jnp.float32),
        pltpu.VMEM((_CCH, 16), jnp.float32),
        pltpu.VMEM((_CCH, 16), jnp.float32),
        pltpu.VMEM((_CCH, D), jnp.float32),
        pltpu.SemaphoreType.DMA,
        pltpu.SemaphoreType.DMA,
    ],
)
def _combine_kernel(y_hbm, pos1_hbm, pos2_hbm, wb1_hbm, wb2_hbm, out_hbm,
                    idx1_v, idx2_v, z1_v, z2_v, wb1_v, wb2_v, o_v, sem1, sem2):
    wid = lax.axis_index("c") * NS + lax.axis_index("s")
    for ch in range(_CN):
        tb = pl.multiple_of(wid * (_CCH * _CN) + ch * _CCH, _CCH)
        pltpu.sync_copy(pos1_hbm.at[pl.ds(tb, _CCH)], idx1_v)
        pltpu.sync_copy(pos2_hbm.at[pl.ds(tb, _CCH)], idx2_v)
        pltpu.sync_copy(wb1_hbm.at[pl.ds(tb, _CCH)], wb1_v)
        pltpu.sync_copy(wb2_hbm.at[pl.ds(tb, _CCH)], wb2_v)
        cp1 = pltpu.async_copy(y_hbm.at[idx1_v], z1_v, sem1)
        cp2 = pltpu.async_copy(y_hbm.at[idx2_v], z2_v, sem2)
        cp1.wait()
        cp2.wait()

        was = [wb1_v[i, pl.ds(0, 16)] for i in range(_CCH)]
        wbs = [wb2_v[i, pl.ds(0, 16)] for i in range(_CCH)]

        def jbody(j, carry):
            for u in range(4):
                off = (j * 4 + u) * 16
                for i in range(_CCH):
                    o_v[i, pl.ds(off, 16)] = (z1_v[i, pl.ds(off, 16)] * was[i]
                                              + z2_v[i, pl.ds(off, 16)] * wbs[i])
            return carry

        lax.fori_loop(0, D // 64, jbody, 0)
        pltpu.sync_copy(o_v, out_hbm.at[pl.ds(tb, _CCH)])


def _gemm_body(tiles, gids, los, his, inits,
               xs_ref, gw_ref, uw_ref, dw_ref, out_ref):
    b = pl.program_id(0)
    j = pl.program_id(1)
    lo = los[b]
    hi = his[b]
    base = tiles[b] * M
    rows = lax.broadcasted_iota(jnp.int32, (M, 1), 0) + base
    mask = jnp.where((rows >= lo) & (rows < hi), 1.0, 0.0)
    x = xs_ref[...]
    gw = gw_ref[0]
    uw = uw_ref[0]
    dw = dw_ref[0]
    gate = lax.dot_general(x, gw, (((1,), (1,)), ((), ())),
                           preferred_element_type=jnp.float32)
    up = lax.dot_general(x, uw, (((1,), (1,)), ((), ())),
                         preferred_element_type=jnp.float32)
    h = (gate * jax.nn.sigmoid(gate)) * up * mask
    part = lax.dot_general(h, dw, (((1,), (1,)), ((), ())),
                           preferred_element_type=jnp.float32)
    first = (inits[b] == 1) & (j == 0)

    @pl.when(first)
    def _():
        out_ref[...] = part

    @pl.when(jnp.logical_not(first))
    def _():
        out_ref[...] += part


def _grouped_gemm(sorted_x, gate_w, up_w, down_w,
                  tiles, gids, los, his, inits):
    grid_spec = pltpu.PrefetchScalarGridSpec(
        num_scalar_prefetch=5,
        grid=(NV, NJ),
        in_specs=[
            pl.BlockSpec((M, D), lambda b, j, tiles, *_: (tiles[b], 0)),
            pl.BlockSpec((1, IB, D), lambda b, j, tiles, gids, *_: (gids[b], j, 0)),
            pl.BlockSpec((1, IB, D), lambda b, j, tiles, gids, *_: (gids[b], j, 0)),
            pl.BlockSpec((1, D, IB), lambda b, j, tiles, gids, *_: (gids[b], 0, j)),
        ],
        out_specs=pl.BlockSpec((M, D), lambda b, j, tiles, *_: (tiles[b], 0)),
    )
    return pl.pallas_call(
        _gemm_body,
        grid_spec=grid_spec,
        out_shape=jax.ShapeDtypeStruct((S, D), jnp.float32),
        compiler_params=pltpu.CompilerParams(
            dimension_semantics=("arbitrary", "arbitrary")),
    )(tiles, gids, los, his, inits, sorted_x, gate_w, up_w, down_w)


def _visit_metadata(offs9):
    t_idx = jnp.arange(NT, dtype=jnp.int32)[:, None]
    lo_tg = jnp.maximum(t_idx * M, offs9[None, :E])
    hi_tg = jnp.minimum((t_idx + 1) * M, offs9[None, 1:])
    nonempty = (lo_tg < hi_tg).reshape(-1)
    order = jnp.cumsum(nonempty.astype(jnp.int32))
    total = order[-1]
    slots = jnp.arange(NV, dtype=jnp.int32)
    pos = jnp.searchsorted(order, slots + 1, side="left").astype(jnp.int32)
    valid = slots < total
    pos = jnp.minimum(pos, NT * E - 1)
    tiles = jnp.where(valid, pos // E, NT - 1)
    gids = jnp.where(valid, pos % E, E - 1)
    los = jnp.where(valid, lo_tg.reshape(-1)[pos], 0)
    his = jnp.where(valid, hi_tg.reshape(-1)[pos], 0)
    prev = jnp.concatenate([jnp.array([-1], jnp.int32), tiles[:-1]])
    inits = (valid & (tiles != prev)).astype(jnp.int32)
    return tiles, gids, los, his, inits


@jax.jit
def _moe(x, router_w, gate_w, up_w, down_w):
    x2 = x.reshape(T, D)
    logits3 = x @ router_w.T
    pos1, pos2, counts, wb1, wb2 = _router(logits3.reshape(T, E))
    pos1 = pos1.reshape(T)
    pos2 = pos2.reshape(T)
    offs9 = jnp.concatenate([jnp.zeros((1,), jnp.int32),
                             jnp.cumsum(counts.reshape(E))]).astype(jnp.int32)

    sorted_x = _dispatch_kernel(x2, pos1, pos2)
    meta = _visit_metadata(offs9)
    y = _grouped_gemm(sorted_x, gate_w, up_w, down_w, *meta)
    out = _combine_kernel(y, pos1, pos2, wb1, wb2)
    return out.reshape(1, T, D), logits3


def kernel(x, router_w, gate_w, up_w, down_w):
    return _moe(x, router_w, gate_w, up_w, down_w)


# speedup vs baseline: 1.2956x; 1.2956x over previous
import functools

import jax
import jax.numpy as jnp
from jax import lax
from jax.experimental import pallas as pl
from jax.experimental.pallas import tpu as pltpu
from jax.experimental.pallas import tpu_sc as plsc

T = 2048
D = 2048
E = 8
TOPK = 2
I = 2048

S = T * TOPK
M = 512
IB = 512
NT = S // M
NV = NT + E - 1
NJ = I // IB
R = 512

_MESH = plsc.VectorSubcoreMesh(core_axis_name="c", subcore_axis_name="s")
NC = 2
NS = 16
NW = NC * NS


def _router_body(logits_ref, pos1_ref, pos2_ref, counts_ref,
                 wb1_ref, wb2_ref):
    logits = logits_ref[...]
    cols = lax.broadcasted_iota(jnp.int32, (T, E), 1)
    m1 = jnp.max(logits, axis=1, keepdims=True)
    i1 = jnp.min(jnp.where(logits == m1, cols, E), axis=1, keepdims=True)
    masked = jnp.where(cols == i1, -jnp.inf, logits)
    m2 = jnp.max(masked, axis=1, keepdims=True)
    i2 = jnp.min(jnp.where(masked == m2, cols, E), axis=1, keepdims=True)
    w1 = 1.0 / (1.0 + jnp.exp(m2 - m1))
    ones16 = jnp.ones((T, 16), jnp.float32)
    wb1_ref[...] = w1 * ones16
    wb2_ref[...] = (1.0 - w1) * ones16

    mA = jnp.where(cols == i1, 1.0, 0.0)
    mB = jnp.where(cols == i2, 1.0, 0.0)
    counts = (jnp.sum(mA, axis=0, keepdims=True)
              + jnp.sum(mB, axis=0, keepdims=True))
    counts_ref[...] = counts.astype(jnp.int32)
    er = lax.broadcasted_iota(jnp.int32, (E, E), 0)
    ec = lax.broadcasted_iota(jnp.int32, (E, E), 1)
    lt_e = jnp.where(er < ec, 1.0, 0.0)
    offs = lax.dot_general(counts, lt_e, (((1,), (0,)), ((), ())),
                           precision=lax.Precision.HIGHEST,
                           preferred_element_type=jnp.float32)

    rr = lax.broadcasted_iota(jnp.int32, (R, R), 0)
    rc = lax.broadcasted_iota(jnp.int32, (R, R), 1)
    ltri = jnp.where(rr > rc, 1.0, 0.0)

    base = jnp.zeros((1, E), jnp.float32)
    for mat, pref in ((mA, pos1_ref), (mB, pos2_ref)):
        for ci in range(T // R):
            mc = lax.slice(mat, (ci * R, 0), (ci * R + R, E))
            prior = lax.dot_general(ltri, mc, (((1,), (0,)), ((), ())),
                                    precision=lax.Precision.HIGHEST,
                                    preferred_element_type=jnp.float32)
            posf = jnp.sum((offs + base + prior) * mc, axis=1, keepdims=True)
            pref[pl.ds(ci * R, R), :] = posf.astype(jnp.int32)
            base = base + jnp.sum(mc, axis=0, keepdims=True)


def _router(logits):
    return pl.pallas_call(
        _router_body,
        out_shape=[
            jax.ShapeDtypeStruct((T, 1), jnp.int32),
            jax.ShapeDtypeStruct((T, 1), jnp.int32),
            jax.ShapeDtypeStruct((1, E), jnp.int32),
            jax.ShapeDtypeStruct((T, 16), jnp.float32),
            jax.ShapeDtypeStruct((T, 16), jnp.float32),
        ],
    )(logits)


_DCH = 32
_DN = T // (NW * _DCH)


@functools.partial(
    pl.kernel,
    mesh=_MESH,
    out_type=jax.ShapeDtypeStruct((S, D), jnp.float32),
    scratch_types=[
        pltpu.VMEM((_DCH,), jnp.int32),
        pltpu.VMEM((_DCH,), jnp.int32),
        pltpu.VMEM((_DCH, D), jnp.float32),
        pltpu.SemaphoreType.DMA,
        pltpu.SemaphoreType.DMA,
    ],
)
def _dispatch_kernel(x_hbm, pos1_hbm, pos2_hbm, out_hbm,
                     idx1_v, idx2_v, rows_v, sem1, sem2):
    wid = lax.axis_index("c") * NS + lax.axis_index("s")
    for ch in range(_DN):
        tb = pl.multiple_of(wid * (_DCH * _DN) + ch * _DCH, _DCH)
        pltpu.sync_copy(pos1_hbm.at[pl.ds(tb, _DCH)], idx1_v)
        pltpu.sync_copy(pos2_hbm.at[pl.ds(tb, _DCH)], idx2_v)
        pltpu.sync_copy(x_hbm.at[pl.ds(tb, _DCH)], rows_v)
        cp1 = pltpu.async_copy(rows_v, out_hbm.at[idx1_v], sem1)
        cp2 = pltpu.async_copy(rows_v, out_hbm.at[idx2_v], sem2)
        cp1.wait()
        cp2.wait()


_CCH = 16
_CN = T // (NW * _CCH)


@functools.partial(
    pl.kernel,
    mesh=_MESH,
    out_type=jax.ShapeDtypeStruct((T, D), jnp.float32),
    scratch_types=[
        pltpu.VMEM((_CCH,), jnp.int32),
        pltpu.VMEM((_CCH,), jnp.int32),
        pltpu.VMEM((_CCH, D), jnp.float32),
        pltpu.VMEM((_CCH, D), jnp.float32),
        pltpu.VMEM((_CCH, 16), jnp.float32),
        pltpu.VMEM((_CCH, 16), jnp.float32),
        pltpu.VMEM((_CCH, D), jnp.float32),
        pltpu.SemaphoreType.DMA,
        pltpu.SemaphoreType.DMA,
    ],
)
def _combine_kernel(y_hbm, pos1_hbm, pos2_hbm, wb1_hbm, wb2_hbm, out_hbm,
                    idx1_v, idx2_v, z1_v, z2_v, wb1_v, wb2_v, o_v, sem1, sem2):
    wid = lax.axis_index("c") * NS + lax.axis_index("s")
    for ch in range(_CN):
        tb = pl.multiple_of(wid * (_CCH * _CN) + ch * _CCH, _CCH)
        pltpu.sync_copy(pos1_hbm.at[pl.ds(tb, _CCH)], idx1_v)
        pltpu.sync_copy(pos2_hbm.at[pl.ds(tb, _CCH)], idx2_v)
        pltpu.sync_copy(wb1_hbm.at[pl.ds(tb, _CCH)], wb1_v)
        pltpu.sync_copy(wb2_hbm.at[pl.ds(tb, _CCH)], wb2_v)
        cp1 = pltpu.async_copy(y_hbm.at[idx1_v], z1_v, sem1)
        cp2 = pltpu.async_copy(y_hbm.at[idx2_v], z2_v, sem2)
        cp1.wait()
        cp2.wait()

        was = [wb1_v[i, pl.ds(0, 16)] for i in range(_CCH)]
        wbs = [wb2_v[i, pl.ds(0, 16)] for i in range(_CCH)]

        def jbody(j, carry):
            for u in range(4):
                off = (j * 4 + u) * 16
                for i in range(_CCH):
                    o_v[i, pl.ds(off, 16)] = (z1_v[i, pl.ds(off, 16)] * was[i]
                                              + z2_v[i, pl.ds(off, 16)] * wbs[i])
            return carry

        lax.fori_loop(0, D // 64, jbody, 0)
        pltpu.sync_copy(o_v, out_hbm.at[pl.ds(tb, _CCH)])


def _gemm_body(tiles, gids, los, his, inits,
               xs_ref, gw_ref, uw_ref, dw_ref, out_ref):
    b = pl.program_id(0)
    j = pl.program_id(1)
    lo = los[b]
    hi = his[b]
    base = tiles[b] * M
    rows = lax.broadcasted_iota(jnp.int32, (M, 1), 0) + base
    mask = jnp.where((rows >= lo) & (rows < hi), 1.0, 0.0)
    x = xs_ref[...]
    gw = gw_ref[0]
    uw = uw_ref[0]
    dw = dw_ref[0]
    gate = lax.dot_general(x, gw, (((1,), (1,)), ((), ())),
                           preferred_element_type=jnp.float32)
    up = lax.dot_general(x, uw, (((1,), (1,)), ((), ())),
                         preferred_element_type=jnp.float32)
    h = (gate * jax.nn.sigmoid(gate)) * up * mask
    part = lax.dot_general(h, dw, (((1,), (1,)), ((), ())),
                           preferred_element_type=jnp.float32)
    first = (inits[b] == 1) & (j == 0)

    @pl.when(first)
    def _():
        out_ref[...] = part

    @pl.when(jnp.logical_not(first))
    def _():
        out_ref[...] += part


def _grouped_gemm(sorted_x, gate_w, up_w, down_w,
                  tiles, gids, los, his, inits):
    grid_spec = pltpu.PrefetchScalarGridSpec(
        num_scalar_prefetch=5,
        grid=(NV, NJ),
        in_specs=[
            pl.BlockSpec((M, D), lambda b, j, tiles, *_: (tiles[b], 0)),
            pl.BlockSpec((1, IB, D), lambda b, j, tiles, gids, *_: (gids[b], j, 0)),
            pl.BlockSpec((1, IB, D), lambda b, j, tiles, gids, *_: (gids[b], j, 0)),
            pl.BlockSpec((1, D, IB), lambda b, j, tiles, gids, *_: (gids[b], 0, j)),
        ],
        out_specs=pl.BlockSpec((M, D), lambda b, j, tiles, *_: (tiles[b], 0)),
    )
    return pl.pallas_call(
        _gemm_body,
        grid_spec=grid_spec,
        out_shape=jax.ShapeDtypeStruct((S, D), jnp.float32),
        compiler_params=pltpu.CompilerParams(
            dimension_semantics=("arbitrary", "arbitrary")),
    )(tiles, gids, los, his, inits, sorted_x, gate_w, up_w, down_w)


def _visit_metadata(offs9):
    t_idx = jnp.arange(NT, dtype=jnp.int32)[:, None]
    lo_tg = jnp.maximum(t_idx * M, offs9[None, :E])
    hi_tg = jnp.minimum((t_idx + 1) * M, offs9[None, 1:])
    nonempty = (lo_tg < hi_tg).reshape(-1)
    order = jnp.cumsum(nonempty.astype(jnp.int32))
    total = order[-1]
    slots = jnp.arange(NV, dtype=jnp.int32)
    pos = jnp.searchsorted(order, slots + 1, side="left").astype(jnp.int32)
    valid = slots < total
    pos = jnp.minimum(pos, NT * E - 1)
    tiles = jnp.where(valid, pos // E, NT - 1)
    gids = jnp.where(valid, pos % E, E - 1)
    los = jnp.where(valid, lo_tg.reshape(-1)[pos], 0)
    his = jnp.where(valid, hi_tg.reshape(-1)[pos], 0)
    prev = jnp.concatenate([jnp.array([-1], jnp.int32), tiles[:-1]])
    inits = (valid & (tiles != prev)).astype(jnp.int32)
    return tiles, gids, los, his, inits


@jax.jit
def _moe(x, router_w, gate_w, up_w, down_w):
    x2 = x.reshape(T, D)
    logits3 = x @ router_w.T
    pos1, pos2, counts, wb1, wb2 = _router(logits3.reshape(T, E))
    pos1 = pos1.reshape(T)
    pos2 = pos2.reshape(T)
    offs9 = jnp.concatenate([jnp.zeros((1,), jnp.int32),
                             jnp.cumsum(counts.reshape(E))]).astype(jnp.int32)

    sorted_x = _dispatch_kernel(x2, pos1, pos2)
    meta = _visit_metadata(offs9)
    y = _grouped_gemm(sorted_x, gate_w, up_w, down_w, *meta)
    out = _combine_kernel(y, pos1, pos2, wb1, wb2)
    return out.reshape(1, T, D), logits3


def kernel(x, router_w, gate_w, up_w, down_w):
    return _moe(x, router_w, gate_w, up_w, down_w)
